# Initial kernel scaffold; baseline (speedup 1.0000x reference)
import jax
import jax.numpy as jnp
from jax.experimental import pallas as pl


def kernel(x, edge_index, W0, b0, g0, be0, W1, b1, g1, be1, W2, b2):
    raise NotImplementedError("write your pallas kernel here")


# speedup vs baseline: 5.0617x; 5.0617x over previous
import functools

import jax
import jax.numpy as jnp
from jax import lax
from jax.experimental import pallas as pl
from jax.experimental.pallas import tpu as pltpu
from jax.experimental.pallas import tpu_sc as plsc

N = 10000
D = 256
H = 128
NC = 2
NS = 16
NW = NC * NS
CH = 128
N_PAD = N + 16
RPS = 624
HR = 640
BM = 1000
EPS = 1e-5


def _sc_degree(dstp):
    n_chunks = dstp.shape[0]
    cpw = n_chunks // NW
    mesh = plsc.VectorSubcoreMesh(core_axis_name="c", subcore_axis_name="s")

    @functools.partial(
        pl.kernel, mesh=mesh,
        out_type=jax.ShapeDtypeStruct((NW, HR * 16), jnp.float32),
        scratch_types=[
            pltpu.VMEM((CH,), jnp.int32),
            pltpu.VMEM((HR * 16,), jnp.float32),
        ],
        compiler_params=pltpu.CompilerParams(needs_layout_passes=False),
    )
    def k(dst_hbm, out_hbm, idxv, hist):
        c = lax.axis_index("c")
        s = lax.axis_index("s")
        w = s * NC + c
        z = jnp.zeros((16,), jnp.float32)

        def zero_body(r, carry):
            hist[pl.ds(r * 16, 16)] = z
            return carry
        lax.fori_loop(0, HR, zero_body, 0)

        ones = jnp.ones((16,), jnp.float32)

        def body(j, carry):
            pltpu.sync_copy(dst_hbm.at[w * cpw + j], idxv)
            for kk in range(CH // 16):
                ii = idxv[pl.ds(kk * 16, 16)]
                plsc.addupdate_scatter(hist, [ii], ones)
            return carry
        lax.fori_loop(0, cpw, body, 0)
        pltpu.sync_copy(hist, out_hbm.at[w])

    return k(dstp)


def _sc_aggregate(hp, srcp, dstp):
    n_chunks = srcp.shape[0]
    cps = n_chunks // NS
    mesh = plsc.VectorSubcoreMesh(core_axis_name="c", subcore_axis_name="s")

    @functools.partial(
        pl.kernel, mesh=mesh,
        out_type=jax.ShapeDtypeStruct((NC, N, H), jnp.float32),
        scratch_types=[
            pltpu.VMEM((CH,), jnp.int32),
            pltpu.VMEM((1, CH), jnp.int32),
            pltpu.VMEM((CH, H), jnp.float32),
            pltpu.VMEM_SHARED((N_PAD, H), jnp.float32),
            pltpu.SemaphoreType.DMA,
        ],
        compiler_params=pltpu.CompilerParams(needs_layout_passes=False),
    )
    def k(hp_hbm, src_hbm, dst_hbm, out_hbm, sidx, didx, rows, acc, sem):
        c = lax.axis_index("c")
        s = lax.axis_index("s")
        r0 = s * RPS
        pltpu.sync_copy(hp_hbm.at[c, pl.ds(r0, RPS)], acc.at[pl.ds(r0, RPS)])

        @pl.when(s == NS - 1)
        def _():
            pltpu.sync_copy(hp_hbm.at[c, pl.ds(NS * RPS, N - NS * RPS)],
                            acc.at[pl.ds(NS * RPS, N - NS * RPS)])
        plsc.subcore_barrier()

        def body(j, carry):
            chunk = s * cps + j
            pltpu.sync_copy(src_hbm.at[chunk], sidx)
            pltpu.sync_copy(dst_hbm.at[chunk], didx.at[0])
            pltpu.async_copy(hp_hbm.at[c].at[sidx], rows, sem).wait()
            pltpu.sync_copy(rows, acc.at[didx.at[0]], add=True)
            return carry
        lax.fori_loop(0, cps, body, 0)

        plsc.subcore_barrier()
        pltpu.sync_copy(acc.at[pl.ds(r0, RPS)], out_hbm.at[c, pl.ds(r0, RPS)])

        @pl.when(s == NS - 1)
        def _():
            pltpu.sync_copy(acc.at[pl.ds(NS * RPS, N - NS * RPS)],
                            out_hbm.at[c, pl.ds(NS * RPS, N - NS * RPS)])

    return k(hp, srcp, dstp)


def _tc_dinv(parts):
    def body(p_ref, o_ref):
        deg = jnp.sum(p_ref[...], axis=0, keepdims=True) + 1.0
        o_ref[...] = lax.rsqrt(deg)
    return pl.pallas_call(
        body,
        out_shape=jax.ShapeDtypeStruct((1, parts.shape[1]), jnp.float32),
    )(parts)


def _split_store(o_ref, hp):
    o_ref[0] = hp[:, :H]
    o_ref[1] = hp[:, H:]


def _tc_pre(x, W, dinv):
    def body(x_ref, w_ref, dv_ref, o_ref):
        h = jnp.dot(x_ref[...], w_ref[...], preferred_element_type=jnp.float32)
        _split_store(o_ref, dv_ref[...] * h)
    grid = N // BM
    return pl.pallas_call(
        body,
        grid=(grid,),
        in_specs=[
            pl.BlockSpec((BM, D), lambda i: (i, 0)),
            pl.BlockSpec((D, D), lambda i: (0, 0)),
            pl.BlockSpec((BM, 1), lambda i: (i, 0)),
        ],
        out_specs=pl.BlockSpec((NC, BM, H), lambda i: (0, i, 0)),
        out_shape=jax.ShapeDtypeStruct((NC, N, H), jnp.float32),
    )(x, W, dinv)


def _combine(s_ref, dv_ref, b_ref):
    seg = jnp.concatenate([s_ref[0], s_ref[1]], axis=1)
    return dv_ref[...] * seg + b_ref[...]


def _tc_stats(segp, dinv, b):
    def body(s_ref, dv_ref, b_ref, o_ref):
        t = _combine(s_ref, dv_ref, b_ref)
        @pl.when(pl.program_id(0) == 0)
        def _():
            o_ref[...] = jnp.zeros_like(o_ref)
        o_ref[0:1, :] += jnp.sum(t, axis=0, keepdims=True)
        o_ref[1:2, :] += jnp.sum(t * t, axis=0, keepdims=True)
    grid = N // BM
    return pl.pallas_call(
        body,
        grid=(grid,),
        in_specs=[
            pl.BlockSpec((NC, BM, H), lambda i: (0, i, 0)),
            pl.BlockSpec((BM, 1), lambda i: (i, 0)),
            pl.BlockSpec((1, D), lambda i: (0, 0)),
        ],
        out_specs=pl.BlockSpec((8, D), lambda i: (0, 0)),
        out_shape=jax.ShapeDtypeStruct((8, D), jnp.float32),
    )(segp, dinv, b)


def _tc_bn_relu_mm(segp, dinv, b, stats, g, be, W):
    def body(s_ref, dv_ref, b_ref, st_ref, g_ref, be_ref, w_ref, o_ref):
        t = _combine(s_ref, dv_ref, b_ref)
        mu = st_ref[0:1, :] * (1.0 / N)
        var = st_ref[1:2, :] * (1.0 / N) - mu * mu
        u = g_ref[...] * (t - mu) * lax.rsqrt(var + EPS) + be_ref[...]
        u = jnp.maximum(u, 0.0)
        h = jnp.dot(u, w_ref[...], preferred_element_type=jnp.float32)
        _split_store(o_ref, dv_ref[...] * h)
    grid = N // BM
    return pl.pallas_call(
        body,
        grid=(grid,),
        in_specs=[
            pl.BlockSpec((NC, BM, H), lambda i: (0, i, 0)),
            pl.BlockSpec((BM, 1), lambda i: (i, 0)),
            pl.BlockSpec((1, D), lambda i: (0, 0)),
            pl.BlockSpec((8, D), lambda i: (0, 0)),
            pl.BlockSpec((1, D), lambda i: (0, 0)),
            pl.BlockSpec((1, D), lambda i: (0, 0)),
            pl.BlockSpec((D, D), lambda i: (0, 0)),
        ],
        out_specs=pl.BlockSpec((NC, BM, H), lambda i: (0, i, 0)),
        out_shape=jax.ShapeDtypeStruct((NC, N, H), jnp.float32),
    )(segp, dinv, b, stats, g, be, W)


def _tc_relu_mm(segp, dinv, b, W):
    def body(s_ref, dv_ref, b_ref, w_ref, o_ref):
        u = jnp.maximum(_combine(s_ref, dv_ref, b_ref), 0.0)
        h = jnp.dot(u, w_ref[...], preferred_element_type=jnp.float32)
        _split_store(o_ref, dv_ref[...] * h)
    grid = N // BM
    return pl.pallas_call(
        body,
        grid=(grid,),
        in_specs=[
            pl.BlockSpec((NC, BM, H), lambda i: (0, i, 0)),
            pl.BlockSpec((BM, 1), lambda i: (i, 0)),
            pl.BlockSpec((1, D), lambda i: (0, 0)),
            pl.BlockSpec((D, D), lambda i: (0, 0)),
        ],
        out_specs=pl.BlockSpec((NC, BM, H), lambda i: (0, i, 0)),
        out_shape=jax.ShapeDtypeStruct((NC, N, H), jnp.float32),
    )(segp, dinv, b, W)


def _tc_post(segp, dinv, b):
    def body(s_ref, dv_ref, b_ref, o_ref):
        o_ref[...] = _combine(s_ref, dv_ref, b_ref)
    grid = N // BM
    return pl.pallas_call(
        body,
        grid=(grid,),
        in_specs=[
            pl.BlockSpec((NC, BM, H), lambda i: (0, i, 0)),
            pl.BlockSpec((BM, 1), lambda i: (i, 0)),
            pl.BlockSpec((1, D), lambda i: (0, 0)),
        ],
        out_specs=pl.BlockSpec((BM, D), lambda i: (i, 0)),
        out_shape=jax.ShapeDtypeStruct((N, D), jnp.float32),
    )(segp, dinv, b)


def kernel(x, edge_index, W0, b0, g0, be0, W1, b1, g1, be1, W2, b2):
    E = edge_index.shape[1]
    epad = -E % (NW * CH)
    src = edge_index[0]
    dst = edge_index[1]
    if epad:
        src = jnp.concatenate([src, jnp.zeros((epad,), jnp.int32)])
        dst = jnp.concatenate([dst, jnp.full((epad,), N, jnp.int32)])
    srcp = src.reshape(-1, CH)
    dstp = dst.reshape(-1, CH)

    parts = _sc_degree(dstp)
    dinv = _tc_dinv(parts).reshape(HR * 16, 1)[:N]

    b0r, g0r, be0r = b0.reshape(1, D), g0.reshape(1, D), be0.reshape(1, D)
    b1r, g1r, be1r = b1.reshape(1, D), g1.reshape(1, D), be1.reshape(1, D)
    b2r = b2.reshape(1, D)

    hp = _tc_pre(x, W0, dinv)
    s = _sc_aggregate(hp, srcp, dstp)
    st = _tc_stats(s, dinv, b0r)
    hp = _tc_bn_relu_mm(s, dinv, b0r, st, g0r, be0r, W1)

    s = _sc_aggregate(hp, srcp, dstp)
    st = _tc_stats(s, dinv, b1r)
    hp = _tc_bn_relu_mm(s, dinv, b1r, st, g1r, be1r, W2)

    s = _sc_aggregate(hp, srcp, dstp)
    hp = _tc_relu_mm(s, dinv, b2r, W2)

    s = _sc_aggregate(hp, srcp, dstp)
    return _tc_post(s, dinv, b2r)


# speedup vs baseline: 7.0004x; 1.3830x over previous
import functools

import jax
import jax.numpy as jnp
from jax import lax
from jax.experimental import pallas as pl
from jax.experimental.pallas import tpu as pltpu
from jax.experimental.pallas import tpu_sc as plsc

N = 10000
D = 256
H = 128
NC = 2
NS = 16
NW = NC * NS
CH = 128
N_PAD = N + 16
RPS = 624
HR = 640
BM = 1000
EPS = 1e-5


def _sc_degree(dstp):
    n_chunks = dstp.shape[0]
    cpw = n_chunks // NW
    mesh = plsc.VectorSubcoreMesh(core_axis_name="c", subcore_axis_name="s")

    @functools.partial(
        pl.kernel, mesh=mesh,
        out_type=jax.ShapeDtypeStruct((NW, HR * 16), jnp.float32),
        scratch_types=[
            pltpu.VMEM((CH,), jnp.int32),
            pltpu.VMEM((HR * 16,), jnp.float32),
        ],
        compiler_params=pltpu.CompilerParams(needs_layout_passes=False),
    )
    def k(dst_hbm, out_hbm, idxv, hist):
        c = lax.axis_index("c")
        s = lax.axis_index("s")
        w = s * NC + c
        z = jnp.zeros((16,), jnp.float32)

        def zero_body(r, carry):
            hist[pl.ds(r * 16, 16)] = z
            return carry
        lax.fori_loop(0, HR, zero_body, 0)

        ones = jnp.ones((16,), jnp.float32)

        def body(j, carry):
            pltpu.sync_copy(dst_hbm.at[w * cpw + j], idxv)
            for kk in range(CH // 16):
                ii = idxv[pl.ds(kk * 16, 16)]
                plsc.addupdate_scatter(hist, [ii], ones)
            return carry
        lax.fori_loop(0, cpw, body, 0)
        pltpu.sync_copy(hist, out_hbm.at[w])

    return k(dstp)


def _sc_aggregate(hp, srcp, dstp):
    n_chunks = srcp.shape[0]
    cps = n_chunks // NS
    mesh = plsc.VectorSubcoreMesh(core_axis_name="c", subcore_axis_name="s")

    @functools.partial(
        pl.kernel, mesh=mesh,
        out_type=jax.ShapeDtypeStruct((NC, N, H), jnp.float32),
        scratch_types=[
            pltpu.VMEM((cps // 2, CH), jnp.int32),
            pltpu.VMEM((cps // 2, CH), jnp.int32),
            pltpu.VMEM((2, CH, H), jnp.float32),
            pltpu.VMEM_SHARED((N_PAD, H), jnp.float32),
            pltpu.SemaphoreType.DMA((2,)),
        ],
        compiler_params=pltpu.CompilerParams(needs_layout_passes=False),
    )
    def k(hp_hbm, src_hbm, dst_hbm, out_hbm, sidx, didx, rows, acc, sem):
        c = lax.axis_index("c")
        s = lax.axis_index("s")
        r0 = s * RPS
        pltpu.sync_copy(hp_hbm.at[c, pl.ds(r0, RPS)], acc.at[pl.ds(r0, RPS)])

        @pl.when(s == NS - 1)
        def _():
            pltpu.sync_copy(hp_hbm.at[c, pl.ds(NS * RPS, N - NS * RPS)],
                            acc.at[pl.ds(NS * RPS, N - NS * RPS)])
        plsc.subcore_barrier()

        def start_gather(j):
            b = lax.rem(j, 2)
            pltpu.async_copy(hp_hbm.at[c].at[sidx.at[j]], rows.at[b],
                             sem.at[b])

        def wait_gather(j):
            b = lax.rem(j, 2)
            pltpu.make_async_copy(hp_hbm.at[c].at[sidx.at[j]], rows.at[b],
                                  sem.at[b]).wait()

        hc = cps // 2
        for p in range(2):
            pltpu.sync_copy(src_hbm.at[pl.ds(s * cps + p * hc, hc)], sidx)
            pltpu.sync_copy(dst_hbm.at[pl.ds(s * cps + p * hc, hc)], didx)
            start_gather(0)

            def body(j, carry):
                @pl.when(j + 1 < hc)
                def _():
                    start_gather(j + 1)
                wait_gather(j)
                pltpu.sync_copy(rows.at[lax.rem(j, 2)], acc.at[didx.at[j]],
                                add=True)
                return carry
            lax.fori_loop(0, hc, body, 0)

        plsc.subcore_barrier()
        pltpu.sync_copy(acc.at[pl.ds(r0, RPS)], out_hbm.at[c, pl.ds(r0, RPS)])

        @pl.when(s == NS - 1)
        def _():
            pltpu.sync_copy(acc.at[pl.ds(NS * RPS, N - NS * RPS)],
                            out_hbm.at[c, pl.ds(NS * RPS, N - NS * RPS)])

    return k(hp, srcp, dstp)


def _tc_dinv(parts):
    def body(p_ref, o_ref):
        deg = jnp.sum(p_ref[...], axis=0, keepdims=True) + 1.0
        o_ref[...] = lax.rsqrt(deg)
    return pl.pallas_call(
        body,
        out_shape=jax.ShapeDtypeStruct((1, parts.shape[1]), jnp.float32),
    )(parts)


def _split_store(o_ref, hp):
    o_ref[0] = hp[:, :H]
    o_ref[1] = hp[:, H:]


def _tc_pre(x, W, dinv):
    def body(x_ref, w_ref, dv_ref, o_ref):
        h = jnp.dot(x_ref[...], w_ref[...], preferred_element_type=jnp.float32)
        _split_store(o_ref, dv_ref[...] * h)
    grid = N // BM
    return pl.pallas_call(
        body,
        grid=(grid,),
        in_specs=[
            pl.BlockSpec((BM, D), lambda i: (i, 0)),
            pl.BlockSpec((D, D), lambda i: (0, 0)),
            pl.BlockSpec((BM, 1), lambda i: (i, 0)),
        ],
        out_specs=pl.BlockSpec((NC, BM, H), lambda i: (0, i, 0)),
        out_shape=jax.ShapeDtypeStruct((NC, N, H), jnp.float32),
    )(x, W, dinv)


def _combine(s_ref, dv_ref, b_ref):
    seg = jnp.concatenate([s_ref[0], s_ref[1]], axis=1)
    return dv_ref[...] * seg + b_ref[...]


def _tc_stats(segp, dinv, b):
    def body(s_ref, dv_ref, b_ref, o_ref):
        t = _combine(s_ref, dv_ref, b_ref)
        @pl.when(pl.program_id(0) == 0)
        def _():
            o_ref[...] = jnp.zeros_like(o_ref)
        o_ref[0:1, :] += jnp.sum(t, axis=0, keepdims=True)
        o_ref[1:2, :] += jnp.sum(t * t, axis=0, keepdims=True)
    grid = N // BM
    return pl.pallas_call(
        body,
        grid=(grid,),
        in_specs=[
            pl.BlockSpec((NC, BM, H), lambda i: (0, i, 0)),
            pl.BlockSpec((BM, 1), lambda i: (i, 0)),
            pl.BlockSpec((1, D), lambda i: (0, 0)),
        ],
        out_specs=pl.BlockSpec((8, D), lambda i: (0, 0)),
        out_shape=jax.ShapeDtypeStruct((8, D), jnp.float32),
    )(segp, dinv, b)


def _tc_bn_relu_mm(segp, dinv, b, stats, g, be, W):
    def body(s_ref, dv_ref, b_ref, st_ref, g_ref, be_ref, w_ref, o_ref):
        t = _combine(s_ref, dv_ref, b_ref)
        mu = st_ref[0:1, :] * (1.0 / N)
        var = st_ref[1:2, :] * (1.0 / N) - mu * mu
        u = g_ref[...] * (t - mu) * lax.rsqrt(var + EPS) + be_ref[...]
        u = jnp.maximum(u, 0.0)
        h = jnp.dot(u, w_ref[...], preferred_element_type=jnp.float32)
        _split_store(o_ref, dv_ref[...] * h)
    grid = N // BM
    return pl.pallas_call(
        body,
        grid=(grid,),
        in_specs=[
            pl.BlockSpec((NC, BM, H), lambda i: (0, i, 0)),
            pl.BlockSpec((BM, 1), lambda i: (i, 0)),
            pl.BlockSpec((1, D), lambda i: (0, 0)),
            pl.BlockSpec((8, D), lambda i: (0, 0)),
            pl.BlockSpec((1, D), lambda i: (0, 0)),
            pl.BlockSpec((1, D), lambda i: (0, 0)),
            pl.BlockSpec((D, D), lambda i: (0, 0)),
        ],
        out_specs=pl.BlockSpec((NC, BM, H), lambda i: (0, i, 0)),
        out_shape=jax.ShapeDtypeStruct((NC, N, H), jnp.float32),
    )(segp, dinv, b, stats, g, be, W)


def _tc_relu_mm(segp, dinv, b, W):
    def body(s_ref, dv_ref, b_ref, w_ref, o_ref):
        u = jnp.maximum(_combine(s_ref, dv_ref, b_ref), 0.0)
        h = jnp.dot(u, w_ref[...], preferred_element_type=jnp.float32)
        _split_store(o_ref, dv_ref[...] * h)
    grid = N // BM
    return pl.pallas_call(
        body,
        grid=(grid,),
        in_specs=[
            pl.BlockSpec((NC, BM, H), lambda i: (0, i, 0)),
            pl.BlockSpec((BM, 1), lambda i: (i, 0)),
            pl.BlockSpec((1, D), lambda i: (0, 0)),
            pl.BlockSpec((D, D), lambda i: (0, 0)),
        ],
        out_specs=pl.BlockSpec((NC, BM, H), lambda i: (0, i, 0)),
        out_shape=jax.ShapeDtypeStruct((NC, N, H), jnp.float32),
    )(segp, dinv, b, W)


def _tc_post(segp, dinv, b):
    def body(s_ref, dv_ref, b_ref, o_ref):
        o_ref[...] = _combine(s_ref, dv_ref, b_ref)
    grid = N // BM
    return pl.pallas_call(
        body,
        grid=(grid,),
        in_specs=[
            pl.BlockSpec((NC, BM, H), lambda i: (0, i, 0)),
            pl.BlockSpec((BM, 1), lambda i: (i, 0)),
            pl.BlockSpec((1, D), lambda i: (0, 0)),
        ],
        out_specs=pl.BlockSpec((BM, D), lambda i: (i, 0)),
        out_shape=jax.ShapeDtypeStruct((N, D), jnp.float32),
    )(segp, dinv, b)


def kernel(x, edge_index, W0, b0, g0, be0, W1, b1, g1, be1, W2, b2):
    E = edge_index.shape[1]
    epad = -E % (NW * CH)
    src = edge_index[0]
    dst = edge_index[1]
    if epad:
        src = jnp.concatenate([src, jnp.zeros((epad,), jnp.int32)])
        dst = jnp.concatenate([dst, jnp.full((epad,), N, jnp.int32)])
    srcp = src.reshape(-1, CH)
    dstp = dst.reshape(-1, CH)

    parts = _sc_degree(dstp)
    dinv = _tc_dinv(parts).reshape(HR * 16, 1)[:N]

    b0r, g0r, be0r = b0.reshape(1, D), g0.reshape(1, D), be0.reshape(1, D)
    b1r, g1r, be1r = b1.reshape(1, D), g1.reshape(1, D), be1.reshape(1, D)
    b2r = b2.reshape(1, D)

    hp = _tc_pre(x, W0, dinv)
    s = _sc_aggregate(hp, srcp, dstp)
    st = _tc_stats(s, dinv, b0r)
    hp = _tc_bn_relu_mm(s, dinv, b0r, st, g0r, be0r, W1)

    s = _sc_aggregate(hp, srcp, dstp)
    st = _tc_stats(s, dinv, b1r)
    hp = _tc_bn_relu_mm(s, dinv, b1r, st, g1r, be1r, W2)

    s = _sc_aggregate(hp, srcp, dstp)
    hp = _tc_relu_mm(s, dinv, b2r, W2)

    s = _sc_aggregate(hp, srcp, dstp)
    return _tc_post(s, dinv, b2r)
